# baseline (speedup 1.0000x reference)
import functools

import jax
import jax.numpy as jnp
from jax import lax
from jax.experimental import pallas as pl
from jax.experimental.pallas import tpu as pltpu
from jax.experimental.pallas import tpu_sc as plsc

_N = 1024
_L = 20
_NW = 32
_BPW = _N // _NW


def _sc_body(item_hbm, hist_hbm, ctx_hbm, out_hbm, iv, hv, cv, obuf, sem):
    wid = lax.axis_index("c") * 16 + lax.axis_index("s")
    base = wid * _BPW

    def per_batch(bb, carry):
        b = base + bb
        pltpu.sync_copy(item_hbm.at[b], iv)
        pltpu.sync_copy(ctx_hbm.at[b], cv)
        pltpu.sync_copy(hist_hbm.at[b], hv)
        item = [iv[i, :] for i in range(3)]
        ctx = [cv[k, :] for k in range(3)]
        for l in range(_L):
            h = [hv[l, j, :] for j in range(3)]
            for g in range(27):
                i, j, k = g // 9, (g // 3) % 3, g % 3
                obuf[l, g, pl.ds(0, 16)] = item[i]
                obuf[l, g, pl.ds(16, 16)] = ctx[k]
                obuf[l, g, pl.ds(32, 16)] = h[j]
        pltpu.sync_copy(obuf, out_hbm.at[b])
        return carry

    lax.fori_loop(0, _BPW, per_batch, 0)


def kernel(x_item, hist, x_context):
    mesh = plsc.VectorSubcoreMesh(core_axis_name="c", subcore_axis_name="s")
    run = functools.partial(
        pl.kernel,
        _sc_body,
        mesh=mesh,
        out_type=jax.ShapeDtypeStruct((_N, _L, 27, 48), jnp.float32),
        scratch_types=[
            pltpu.VMEM((3, 16), jnp.float32),
            pltpu.VMEM((_L, 3, 16), jnp.float32),
            pltpu.VMEM((3, 16), jnp.float32),
            pltpu.VMEM((_L, 27, 48), jnp.float32),
            pltpu.SemaphoreType.DMA,
        ],
    )()
    return run(x_item, hist, x_context)


# speedup vs baseline: 2.4673x; 2.4673x over previous
import functools

import jax
import jax.numpy as jnp
from jax import lax
from jax.experimental import pallas as pl
from jax.experimental.pallas import tpu as pltpu
from jax.experimental.pallas import tpu_sc as plsc

_N = 1024
_L = 20
_NW = 32
_BPW = _N // _NW
_ROW = 27 * 48
_BLK = _L * _ROW


def _assemble(iv, hv, cv, ob):
    item = [iv[pl.ds(16 * i, 16)] for i in range(3)]
    ctx = [cv[pl.ds(16 * k, 16)] for k in range(3)]
    for l in range(_L):
        h = [hv[pl.ds(48 * l + 16 * j, 16)] for j in range(3)]
        for g in range(27):
            i, j, k = g // 9, (g // 3) % 3, g % 3
            o = l * _ROW + g * 48
            ob[pl.ds(o, 16)] = item[i]
            ob[pl.ds(o + 16, 16)] = ctx[k]
            ob[pl.ds(o + 32, 16)] = h[j]


def _sc_body(item_hbm, hist_hbm, ctx_hbm, out_hbm,
             iv0, hv0, cv0, iv1, hv1, cv1, ob0, ob1,
             si0, si1, so0, so1):
    wid = lax.axis_index("c") * 16 + lax.axis_index("s")
    base = wid * _BPW
    ins = [(iv0, hv0, cv0, si0), (iv1, hv1, cv1, si1)]
    outs = [(ob0, so0), (ob1, so1)]

    def fetch(b, slot):
        iv, hv, cv, si = ins[slot]
        pltpu.async_copy(item_hbm.at[b], iv, si)
        pltpu.async_copy(hist_hbm.at[b], hv, si)
        pltpu.async_copy(ctx_hbm.at[b], cv, si)

    def fetch_wait(b, slot):
        iv, hv, cv, si = ins[slot]
        pltpu.make_async_copy(item_hbm.at[b], iv, si).wait()
        pltpu.make_async_copy(hist_hbm.at[b], hv, si).wait()
        pltpu.make_async_copy(ctx_hbm.at[b], cv, si).wait()

    fetch(base, 0)
    fetch(base + 1, 1)

    def step(p, slot, first):
        b = base + 2 * p + slot
        iv, hv, cv, _ = ins[slot]
        ob, so = outs[slot]
        fetch_wait(b, slot)
        if first is not None:
            first()
        else:
            pltpu.make_async_copy(ob, out_hbm.at[b], so).wait()
        _assemble(iv, hv, cv, ob)
        pltpu.async_copy(ob, out_hbm.at[b], so)

    step(0, 0, lambda: None)
    fetch(base + 2, 0)
    step(0, 1, lambda: None)
    fetch(base + 3, 1)

    def pair(p, carry):
        step(p, 0, None)

        @pl.when(p < _BPW // 2 - 1)
        def _():
            fetch(base + 2 * p + 2, 0)

        step(p, 1, None)

        @pl.when(p < _BPW // 2 - 1)
        def _():
            fetch(base + 2 * p + 3, 1)

        return carry

    lax.fori_loop(1, _BPW // 2, pair, 0)
    pltpu.make_async_copy(ob0, out_hbm.at[base], so0).wait()
    pltpu.make_async_copy(ob1, out_hbm.at[base + 1], so1).wait()


def kernel(x_item, hist, x_context):
    mesh = plsc.VectorSubcoreMesh(core_axis_name="c", subcore_axis_name="s")
    run = functools.partial(
        pl.kernel,
        _sc_body,
        mesh=mesh,
        out_type=jax.ShapeDtypeStruct((_N, _BLK), jnp.float32),
        scratch_types=[
            pltpu.VMEM((48,), jnp.float32),
            pltpu.VMEM((_L * 48,), jnp.float32),
            pltpu.VMEM((48,), jnp.float32),
            pltpu.VMEM((48,), jnp.float32),
            pltpu.VMEM((_L * 48,), jnp.float32),
            pltpu.VMEM((48,), jnp.float32),
            pltpu.VMEM((_BLK,), jnp.float32),
            pltpu.VMEM((_BLK,), jnp.float32),
            pltpu.SemaphoreType.DMA,
            pltpu.SemaphoreType.DMA,
            pltpu.SemaphoreType.DMA,
            pltpu.SemaphoreType.DMA,
        ],
    )()
    flat = run(x_item.reshape(_N, 48), hist.reshape(_N, _L * 48),
               x_context.reshape(_N, 48))
    return flat.reshape(_N, _L, 27, 48)


# speedup vs baseline: 6.5033x; 2.6359x over previous
import functools

import jax
import jax.numpy as jnp
from jax import lax
from jax.experimental import pallas as pl
from jax.experimental.pallas import tpu as pltpu
from jax.experimental.pallas import tpu_sc as plsc

_N = 1024
_L = 20
_NW = 32
_PAIRS = _L * 27
_PPW = 17
_WIN = 6


def _sc_body(item_hbm, hist_hbm, ctx_hbm, out_hbm,
             item_s, hist_s, ctx_s, sem_in, sem_out):
    cid = lax.axis_index("c")
    sid = lax.axis_index("s")
    wid = cid * 16 + sid

    @pl.when(sid < 15)
    def _():
        r0 = sid * 64
        pltpu.async_copy(hist_hbm.at[pl.ds(r0, 64)],
                         hist_s.at[pl.ds(r0, 64)], sem_in)
        pltpu.make_async_copy(hist_hbm.at[pl.ds(r0, 64)],
                              hist_s.at[pl.ds(r0, 64)], sem_in).wait()

    @pl.when(sid == 15)
    def _():
        pltpu.async_copy(item_hbm, item_s, sem_in)
        pltpu.async_copy(ctx_hbm, ctx_s, sem_in)
        pltpu.make_async_copy(item_hbm, item_s, sem_in).wait()
        pltpu.make_async_copy(ctx_hbm, ctx_s, sem_in).wait()

    plsc.subcore_barrier()

    start = wid * _PPW - lax.max(wid - 28, 0)
    count = jnp.where(wid < 28, _PPW, _PPW - 1)

    def copies(p):
        l = p // 27
        g = p % 27
        i = g // 9
        j = (g % 9) // 3
        k = g % 3
        r0 = l * 1296 + g * 48
        return (
            pltpu.make_async_copy(item_s.at[pl.ds(i * 16, 16)],
                                  out_hbm.at[pl.ds(r0, 16)], sem_out),
            pltpu.make_async_copy(ctx_s.at[pl.ds(k * 16, 16)],
                                  out_hbm.at[pl.ds(r0 + 16, 16)], sem_out),
            pltpu.make_async_copy(hist_s.at[pl.ds(l * 48 + j * 16, 16)],
                                  out_hbm.at[pl.ds(r0 + 32, 16)], sem_out),
        )

    def drain_one():
        d = pltpu.make_async_copy(item_s.at[pl.ds(0, 16)],
                                  out_hbm.at[pl.ds(0, 16)], sem_out)
        d.wait()
        d.wait()
        d.wait()

    def body(it, carry):
        @pl.when(it < count)
        def _():
            a, b, c = copies(start + it)
            a.start()
            b.start()
            c.start()

        @pl.when((it >= _WIN) & (it - _WIN < count))
        def _():
            drain_one()

        return carry

    lax.fori_loop(0, _PPW, body, 0)

    rem = count - (_PPW - _WIN)

    def tail(it, carry):
        @pl.when(it < rem)
        def _():
            drain_one()

        return carry

    lax.fori_loop(0, _WIN, tail, 0)


def kernel(x_item, hist, x_context):
    mesh = plsc.VectorSubcoreMesh(core_axis_name="c", subcore_axis_name="s")
    run = functools.partial(
        pl.kernel,
        _sc_body,
        mesh=mesh,
        out_type=jax.ShapeDtypeStruct((_PAIRS * 48, _N), jnp.float32),
        scratch_types=[
            pltpu.VMEM_SHARED((48, _N), jnp.float32),
            pltpu.VMEM_SHARED((960, _N), jnp.float32),
            pltpu.VMEM_SHARED((48, _N), jnp.float32),
            pltpu.SemaphoreType.DMA,
            pltpu.SemaphoreType.DMA,
        ],
    )()
    flat = run(x_item.reshape(_N, 48).T, hist.reshape(_N, 960).T,
               x_context.reshape(_N, 48).T)
    return jnp.swapaxes(flat, 0, 1).reshape(_N, _L, 27, 48)
